# baseline (device time: 22784 ns/iter reference)
import jax
import jax.numpy as jnp
from jax import lax
from jax.experimental import pallas as pl
from jax.experimental.pallas import tpu as pltpu

N_CHUNKS = 8


def kernel(x):
    _, m, n_full = x.shape
    n_half = n_full // 2
    m_half = m // 2
    rc = m_half // N_CHUNKS

    def body(
        x_hbm, out_ref, ybuf, kbuf, sbuf,
        stage_sem, local_sems,
        ysend_sems, yrecv_sems, xsend_sems, xrecv_sems,
    ):
        my_x = lax.axis_index("x")
        my_y = lax.axis_index("y")
        y_peer = (my_x, 1 - my_y)
        x_peer = (1 - my_x, my_y)

        row_base = my_x * m_half
        col_send = (1 - my_y) * n_half
        col_keep = my_y * n_half

        local_cps = []
        for k in range(N_CHUNKS):
            cp = pltpu.make_async_copy(
                x_hbm.at[0, pl.ds(row_base + k * rc, rc),
                         pl.ds(col_keep, n_half)],
                kbuf.at[pl.ds(k * rc, rc), :],
                local_sems.at[k],
            )
            cp.start()
            local_cps.append(cp)

        s0 = pltpu.make_async_copy(
            x_hbm.at[0, pl.ds(row_base, rc), pl.ds(col_send, n_half)],
            sbuf, stage_sem,
        )
        s0.start()

        barrier_sem = pltpu.get_barrier_semaphore()
        for nbr in (y_peer, x_peer):
            pl.semaphore_signal(
                barrier_sem, inc=1, device_id=nbr,
                device_id_type=pl.DeviceIdType.MESH,
            )
        pl.semaphore_wait(barrier_sem, 2)

        s0.wait()
        y_rdmas = []
        for k in range(N_CHUNKS):
            src = (sbuf if k == 0 else
                   x_hbm.at[0, pl.ds(row_base + k * rc, rc),
                            pl.ds(col_send, n_half)])
            rdma = pltpu.make_async_remote_copy(
                src_ref=src,
                dst_ref=ybuf.at[pl.ds(k * rc, rc), :],
                send_sem=ysend_sems.at[k],
                recv_sem=yrecv_sems.at[k],
                device_id=y_peer,
                device_id_type=pl.DeviceIdType.MESH,
            )
            rdma.start()
            y_rdmas.append(rdma)

        x_rdmas = []
        out_cps = []
        for k in range(N_CHUNKS):
            y_rdmas[k].wait_recv()
            local_cps[k].wait()
            ch = pl.ds(k * rc, rc)
            rows = pl.ds(row_base + k * rc, rc)
            out_ref[rows, :] = kbuf[ch, :] + ybuf[ch, :]
            rdma = pltpu.make_async_remote_copy(
                src_ref=out_ref.at[rows, :],
                dst_ref=out_ref.at[rows, :],
                send_sem=xsend_sems.at[k],
                recv_sem=xrecv_sems.at[k],
                device_id=x_peer,
                device_id_type=pl.DeviceIdType.MESH,
            )
            rdma.start()
            x_rdmas.append(rdma)

        for k in range(N_CHUNKS):
            x_rdmas[k].wait_recv()
        for k in range(N_CHUNKS):
            y_rdmas[k].wait_send()
            x_rdmas[k].wait_send()

    return pl.pallas_call(
        body,
        out_shape=jax.ShapeDtypeStruct((m, n_half), jnp.float32),
        in_specs=[pl.BlockSpec(memory_space=pl.ANY)],
        out_specs=pl.BlockSpec(memory_space=pltpu.VMEM),
        scratch_shapes=[
            pltpu.VMEM((m_half, n_half), jnp.float32),
            pltpu.VMEM((m_half, n_half), jnp.float32),
            pltpu.VMEM((m // 2 // N_CHUNKS, n_half), jnp.float32),
            pltpu.SemaphoreType.DMA,
            pltpu.SemaphoreType.DMA((N_CHUNKS,)),
            pltpu.SemaphoreType.DMA((N_CHUNKS,)),
            pltpu.SemaphoreType.DMA((N_CHUNKS,)),
            pltpu.SemaphoreType.DMA((N_CHUNKS,)),
            pltpu.SemaphoreType.DMA((N_CHUNKS,)),
        ],
        compiler_params=pltpu.CompilerParams(collective_id=0),
    )(x)
